# Initial kernel scaffold; baseline (speedup 1.0000x reference)
import jax
import jax.numpy as jnp
from jax.experimental import pallas as pl


def kernel(x, W_enc, b_enc, W_dec_q, b_dec_q, W_dec_e, b_dec_e, embeddings):
    raise NotImplementedError("write your pallas kernel here")


# speedup vs baseline: 2.7002x; 2.7002x over previous
import functools

import jax
import jax.numpy as jnp
from jax import lax
from jax.experimental import pallas as pl
from jax.experimental.pallas import tpu as pltpu
from jax.experimental.pallas import tpu_sc as plsc

B = 1024
D_IN = 512
LATENT = 64
SOM_H = 32
SOM_W = 32
N_CODES = SOM_H * SOM_W
PAD_ROWS = 8
ZROW = N_CODES
LATP = 128
BM = 128
GRID = B // BM

_DOT = functools.partial(
    jnp.dot,
    preferred_element_type=jnp.float32,
    precision=lax.Precision.DEFAULT,
)
_DOT_HI = functools.partial(
    jnp.dot,
    preferred_element_type=jnp.float32,
    precision=lax.Precision.HIGHEST,
)


def _tc_body(x_ref, we_ref, be_ref, wdq_ref, bdq_ref, wde_ref, bde_ref,
             e_ref, et_ref, xe_ref, ze_ref, dist_ref, k_ref, cb_ref,
             e128_ref):
    i = pl.program_id(0)
    z = _DOT(x_ref[:], we_ref[:]) + be_ref[:]
    ze_ref[:] = z
    xe_ref[:] = _DOT(z, wde_ref[:]) + bde_ref[:]
    score = _DOT_HI(z, et_ref[:])
    ee = jnp.sum(et_ref[:] * et_ref[:], axis=0, keepdims=True)
    key = ee - 2.0 * score
    zz = jnp.sum(z * z, axis=1, keepdims=True)
    dist_ref[:] = key + zz
    m = jnp.min(key, axis=1, keepdims=True)
    iot = lax.broadcasted_iota(jnp.int32, (BM, N_CODES), 1)
    k_ref[:] = jnp.min(jnp.where(key == m, iot, jnp.int32(N_CODES)),
                       axis=1, keepdims=True)

    @pl.when(i == 0)
    def _():
        cb = _DOT(e_ref[:], wdq_ref[:]) + bdq_ref[:]
        cb_ref[0:N_CODES, :] = cb
        cb_ref[N_CODES:, :] = jnp.zeros((PAD_ROWS, D_IN), jnp.float32)
        e128_ref[:] = jnp.zeros((N_CODES + PAD_ROWS, LATP), jnp.float32)
        e128_ref[0:N_CODES, 0:LATENT] = e_ref[:]


def _tc_forward(x, W_enc, b_enc, W_dec_q, b_dec_q, W_dec_e, b_dec_e, e2d, et):
    full = lambda r, c: pl.BlockSpec((r, c), lambda i: (0, 0))
    return pl.pallas_call(
        _tc_body,
        grid=(GRID,),
        in_specs=[
            pl.BlockSpec((BM, D_IN), lambda i: (i, 0)),
            full(D_IN, LATENT),
            full(1, LATENT),
            full(LATENT, D_IN),
            full(1, D_IN),
            full(LATENT, D_IN),
            full(1, D_IN),
            full(N_CODES, LATENT),
            full(LATENT, N_CODES),
        ],
        out_specs=[
            pl.BlockSpec((BM, D_IN), lambda i: (i, 0)),
            pl.BlockSpec((BM, LATENT), lambda i: (i, 0)),
            pl.BlockSpec((BM, N_CODES), lambda i: (i, 0)),
            pl.BlockSpec((BM, 1), lambda i: (i, 0)),
            full(N_CODES + PAD_ROWS, D_IN),
            full(N_CODES + PAD_ROWS, LATP),
        ],
        out_shape=[
            jax.ShapeDtypeStruct((B, D_IN), jnp.float32),
            jax.ShapeDtypeStruct((B, LATENT), jnp.float32),
            jax.ShapeDtypeStruct((B, N_CODES), jnp.float32),
            jax.ShapeDtypeStruct((B, 1), jnp.int32),
            jax.ShapeDtypeStruct((N_CODES + PAD_ROWS, D_IN), jnp.float32),
            jax.ShapeDtypeStruct((N_CODES + PAD_ROWS, LATP), jnp.float32),
        ],
    )(x, W_enc, b_enc, W_dec_q, b_dec_q, W_dec_e, b_dec_e, e2d, et)


_NC = 2
_NS = 16
_NW = _NC * _NS
BPW = B // _NW


def _sc_body(k_hbm, e_hbm, cb_hbm, zq_hbm, up_hbm, dn_hbm, lf_hbm, xq_hbm,
             kv, uv, dv, lv, rows, cbrows, sem):
    wid = lax.axis_index("s") * _NC + lax.axis_index("c")
    base = wid * BPW
    pltpu.sync_copy(k_hbm.at[pl.ds(base, BPW)], kv)
    for c in range(BPW // 16):
        kk = kv[pl.ds(c * 16, 16)]
        k1 = kk >> 5
        k2 = kk & 31
        zrow = jnp.full((16,), ZROW, jnp.int32)
        uv[pl.ds(c * 16, 16)] = jnp.where(k1 < SOM_H - 1, kk + SOM_W, zrow)
        dv[pl.ds(c * 16, 16)] = jnp.where(k1 > 0, kk - SOM_W, zrow)
        lv[pl.ds(c * 16, 16)] = jnp.where(k2 > 0, kk - 1, zrow)
    pltpu.async_copy(e_hbm.at[kv], rows, sem).wait()
    pltpu.sync_copy(rows, zq_hbm.at[pl.ds(base, BPW)])
    pltpu.async_copy(e_hbm.at[uv], rows, sem).wait()
    pltpu.sync_copy(rows, up_hbm.at[pl.ds(base, BPW)])
    pltpu.async_copy(e_hbm.at[dv], rows, sem).wait()
    pltpu.sync_copy(rows, dn_hbm.at[pl.ds(base, BPW)])
    pltpu.async_copy(e_hbm.at[lv], rows, sem).wait()
    pltpu.sync_copy(rows, lf_hbm.at[pl.ds(base, BPW)])
    pltpu.async_copy(cb_hbm.at[kv], cbrows, sem).wait()
    pltpu.sync_copy(cbrows, xq_hbm.at[pl.ds(base, BPW)])


@functools.lru_cache(maxsize=1)
def _make_sc_gather():
    return functools.partial(
        pl.kernel,
        out_type=[
            jax.ShapeDtypeStruct((B, LATP), jnp.float32),
            jax.ShapeDtypeStruct((B, LATP), jnp.float32),
            jax.ShapeDtypeStruct((B, LATP), jnp.float32),
            jax.ShapeDtypeStruct((B, LATP), jnp.float32),
            jax.ShapeDtypeStruct((B, D_IN), jnp.float32),
        ],
        scratch_types=[
            pltpu.VMEM((BPW,), jnp.int32),
            pltpu.VMEM((BPW,), jnp.int32),
            pltpu.VMEM((BPW,), jnp.int32),
            pltpu.VMEM((BPW,), jnp.int32),
            pltpu.VMEM((BPW, LATP), jnp.float32),
            pltpu.VMEM((BPW, D_IN), jnp.float32),
            pltpu.SemaphoreType.DMA,
        ],
        mesh=plsc.VectorSubcoreMesh(core_axis_name="c", subcore_axis_name="s"),
    )(_sc_body)


def kernel(x, W_enc, b_enc, W_dec_q, b_dec_q, W_dec_e, b_dec_e, embeddings):
    e2d = embeddings.reshape(N_CODES, LATENT)
    et = e2d.T
    x_e, z_e, z_dist_flat, k2d, cb, e128 = _tc_forward(
        x, W_enc, b_enc.reshape(1, LATENT),
        W_dec_q, b_dec_q.reshape(1, D_IN),
        W_dec_e, b_dec_e.reshape(1, D_IN), e2d, et)
    k = k2d.reshape(B)
    zq_p, up_p, dn_p, lf_p, x_q = _make_sc_gather()(k, e128, cb)
    z_q = zq_p[:, :LATENT]
    z_q_right = jnp.zeros_like(z_q)
    z_q_neighbors = jnp.stack(
        [z_q, up_p[:, :LATENT], dn_p[:, :LATENT], z_q_right,
         lf_p[:, :LATENT]], axis=1)
    return (x_e, x_q, z_e, z_q, z_q_neighbors, k, z_dist_flat)
